# baseline (speedup 1.0000x reference)
import functools
import math

import jax
import jax.numpy as jnp
from jax import lax
from jax.experimental import pallas as pl
from jax.experimental.pallas import tpu as pltpu
from jax.experimental.pallas import tpu_sc as plsc

SEL = 64
N_ITER = 4
NEG = -3.0e38


def _sums_body(q_ref, k_ref, out_ref):
    ones = jnp.ones((64, 64), jnp.float32)
    ks = jax.lax.dot_general(ones, k_ref[0, 0], (((1,), (1,)), ((), ())),
                             precision=jax.lax.Precision.HIGH,
                             preferred_element_type=jnp.float32)
    qs = jax.lax.dot_general(ones, q_ref[0, 0], (((1,), (1,)), ((), ())),
                             precision=jax.lax.Precision.HIGH,
                             preferred_element_type=jnp.float32)
    out_ref[0, 0, 0, :] = ks[0]
    out_ref[0, 0, 1, :] = qs[0]


def _select_body(s_ref, idx_ref):
    R, N = s_ref.shape
    vals0 = s_ref[...]
    lane = jax.lax.broadcasted_iota(jnp.int32, (R, N), 1)
    col = jax.lax.broadcasted_iota(jnp.int32, (R, SEL), 1)

    def body(i, carry):
        vals, acc = carry
        idx = jnp.argmax(vals, axis=-1).astype(jnp.int32)
        acc = jnp.where(col == i, idx[:, None], acc)
        vals = jnp.where(lane == idx[:, None], NEG, vals)
        return vals, acc

    _, acc = jax.lax.fori_loop(
        0, SEL, body, (vals0, jnp.zeros((R, SEL), jnp.int32)))
    idx_ref[...] = acc


CH = 512
NCHUNK = 4096 // CH


def _sc_body(q_hbm, k_hbm, out_hbm, buf0, buf1, sums, orig, marr, lista,
             listb, final, sem0, sem1):
    w = lax.axis_index("s") * 2 + lax.axis_index("c")
    b = w // 16
    h = w % 16
    iota = lax.iota(jnp.int32, 16)
    fiota = iota.astype(jnp.float32) * 0.0
    bufs = (buf0, buf1)
    sems = (sem0, sem1)

    def chunk_sums(ch, buf):
        def group_body(g, _):
            rowbase = (g * 16 + iota) * 64

            def d_body(d8, acc):
                g0 = [plsc.load_gather(buf, [rowbase + (d8 * 8 + t)])
                      for t in range(8)]
                return acc + (((g0[0] + g0[1]) + (g0[2] + g0[3]))
                              + ((g0[4] + g0[5]) + (g0[6] + g0[7])))

            acc = lax.fori_loop(0, 8, d_body, fiota)
            sums[pl.ds(ch * CH + g * 16, 16)] = acc
            return 0

        lax.fori_loop(0, CH // 16, group_body, 0)

    for src, hbm in ((0, k_hbm), (1, q_hbm)):
        cps = [None] * NCHUNK
        for ch in range(2):
            cps[ch] = pltpu.make_async_copy(
                hbm.at[b, h, pl.ds(ch * CH * 64, CH * 64)],
                bufs[ch % 2], sems[ch % 2])
            cps[ch].start()
        for ch in range(NCHUNK):
            cps[ch].wait()
            if ch + 2 < NCHUNK:
                cps[ch + 2] = pltpu.make_async_copy(
                    hbm.at[b, h, pl.ds((ch + 2) * CH * 64, CH * 64)],
                    bufs[ch % 2], sems[ch % 2])
            chunk_sums(ch, bufs[ch % 2])
            if ch + 2 < NCHUNK:
                cps[ch + 2].start()

        def copy_body(v, _):
            orig[pl.ds(v * 16, 16)] = sums[pl.ds(v * 16, 16)]
            return 0
        lax.fori_loop(0, 256, copy_body, 0)

        def mbuild(g, _):
            def mb(i, m):
                return jnp.maximum(m, sums[pl.ds(g * 256 + i * 16, 16)])
            marr[pl.ds(g * 16, 16)] = lax.fori_loop(0, 16, mb, fiota + NEG)
            return 0
        lax.fori_loop(0, 16, mbuild, 0)

        def extract(t, s_last):
            mv = [marr[pl.ds(g * 16, 16)] for g in range(16)]
            mm = mv[0]
            for g in range(1, 16):
                mm = jnp.maximum(mm, mv[g])
            gmax = jnp.max(mm)
            lsel = plsc.all_reduce_ffs(mm == gmax)
            colv = plsc.load_gather(marr, [iota * 16 + lsel])
            gsel = plsc.all_reduce_ffs(colv == gmax)
            cidx = gsel * 256 + iota * 16 + lsel
            gath = plsc.load_gather(sums, [cidx])
            isel = plsc.all_reduce_ffs(gath == gmax)
            gidx = gsel * 256 + isel * 16 + lsel
            plsc.store_scatter(sums, [jnp.broadcast_to(gidx, (16,))],
                               fiota + NEG, mask=iota == 0)
            gath2 = plsc.load_gather(sums, [cidx])
            newm = jnp.max(gath2)
            gsel_s = jnp.max(gsel)
            mrow = marr[pl.ds(gsel_s * 16, 16)]
            marr[pl.ds(gsel_s * 16, 16)] = jnp.where(iota == lsel, newm, mrow)
            return gmax

        s64 = lax.fori_loop(0, SEL, extract, jnp.float32(NEG))

        def gather_lists(v, carry):
            na, nb = carry
            x = orig[pl.ds(v * 16, 16)]
            idxv = v * 16 + iota
            gt = x > s64
            eq = x == s64
            plsc.store_compressed(lista.at[pl.ds(na, 16)], idxv, mask=gt)
            plsc.store_compressed(listb.at[pl.ds(nb, 16)], idxv, mask=eq)
            na = na + jnp.max(plsc.all_reduce_population_count(gt))
            nb = nb + jnp.max(plsc.all_reduce_population_count(eq))
            return na, nb

        n1, _ = lax.fori_loop(0, 256, gather_lists,
                              (jnp.int32(0), jnp.int32(0)))

        for g in range(4):
            final[pl.ds(g * 16, 16)] = lista[pl.ds(g * 16, 16)]
        for g in range(4):
            listbv = listb[pl.ds(g * 16, 16)]
            final[pl.ds(n1 + g * 16, 16)] = listbv

        pltpu.sync_copy(final.at[pl.ds(0, 128)], out_hbm.at[2 * w + src])


def _sc_select(Q, K):
    kern = pl.kernel(
        _sc_body,
        out_type=jax.ShapeDtypeStruct((2 * 32, 128), jnp.int32),
        mesh=plsc.VectorSubcoreMesh(core_axis_name="c", subcore_axis_name="s"),
        compiler_params=pltpu.CompilerParams(needs_layout_passes=False),
        scratch_types=[
            pltpu.VMEM((CH * 64,), jnp.float32),
            pltpu.VMEM((CH * 64,), jnp.float32),
            pltpu.VMEM((4096,), jnp.float32),
            pltpu.VMEM((4096,), jnp.float32),
            pltpu.VMEM((256,), jnp.float32),
            pltpu.VMEM((4160,), jnp.int32),
            pltpu.VMEM((4160,), jnp.int32),
            pltpu.VMEM((192,), jnp.int32),
            pltpu.SemaphoreType.DMA,
            pltpu.SemaphoreType.DMA,
        ],
    )
    B, H, N, D = Q.shape
    return kern(Q.reshape(B, H, N * D), K.reshape(B, H, N * D))[:, :SEL]


def _mm(a, b, ca, cb):
    return jax.lax.dot_general(
        a, b, ((( ca,), (cb,)), ((), ())),
        preferred_element_type=jnp.float32)


def _head(qs, k, v, idx_k, idx_q):
    N = qs.shape[0]
    lane = jax.lax.broadcasted_iota(jnp.int32, (SEL, N), 1)
    oh_k = (lane == idx_k[:, None]).astype(jnp.float32)
    oh_q = (lane == idx_q[:, None]).astype(jnp.float32)

    nc = _mm(oh_k, k, 1, 0)
    nr = _mm(oh_q, qs, 1, 0)

    c = _mm(qs, nc, 1, 1)
    E = jnp.exp(c - jnp.max(c, axis=-1, keepdims=True))
    S = jnp.sum(E, axis=-1, keepdims=True)

    cu = _mm(nr, nc, 1, 1)
    eu = jnp.exp(cu - jnp.max(cu, axis=-1, keepdims=True))
    u = eu / jnp.sum(eu, axis=-1, keepdims=True)

    r = _mm(nr, k, 1, 1)
    er = jnp.exp(r - jnp.max(r, axis=-1, keepdims=True))
    k3 = er / jnp.sum(er, axis=-1, keepdims=True)

    rv = _mm(k3, v, 1, 0)

    eye = (jax.lax.broadcasted_iota(jnp.int32, (SEL, SEL), 0)
           == jax.lax.broadcasted_iota(jnp.int32, (SEL, SEL), 1)
           ).astype(jnp.float32)
    denom = jnp.max(jnp.sum(u, axis=0))
    vinv = _mm(u, eye, 0, 0) * (1.0 / denom)
    for _ in range(N_ITER):
        kv = _mm(u, vinv, 1, 0)
        a1 = 7.0 * eye - kv
        a2 = 15.0 * eye - _mm(kv, a1, 1, 0)
        a3 = 13.0 * eye - _mm(kv, a2, 1, 0)
        vinv = 0.25 * _mm(vinv, a3, 1, 0)

    m = _mm(vinv, rv, 1, 0)
    return _mm(E, m, 1, 0) / S


def _main_body(q_ref, k_ref, v_ref, ik_ref, iq_ref, out_ref):
    scale = jnp.float32(1.0 / math.sqrt(64.0))
    for j in range(q_ref.shape[1]):
        out_ref[0, j] = _head(q_ref[0, j] * scale, k_ref[0, j], v_ref[0, j],
                              ik_ref[0, j], iq_ref[0, j])


HB = 2


def kernel(Q, K, V, mask):
    B, H, N, D = Q.shape
    BH = B * H

    idx = _sc_select(Q, K)

    idx = idx.reshape(B * H // HB, HB, 2, SEL)
    idx_k = idx[:, :, 0, :]
    idx_q = idx[:, :, 1, :]

    HBLK = H // HB
    X = pl.pallas_call(
        _main_body,
        grid=(BH // HB,),
        in_specs=[
            pl.BlockSpec((1, HB, N, D), lambda i: (i // HBLK, i % HBLK, 0, 0)),
            pl.BlockSpec((1, HB, N, D), lambda i: (i // HBLK, i % HBLK, 0, 0)),
            pl.BlockSpec((1, HB, N, D), lambda i: (i // HBLK, i % HBLK, 0, 0)),
            pl.BlockSpec((1, HB, SEL), lambda i: (i, 0, 0)),
            pl.BlockSpec((1, HB, SEL), lambda i: (i, 0, 0)),
        ],
        out_specs=pl.BlockSpec((1, HB, N, D), lambda i: (i // HBLK, i % HBLK, 0, 0)),
        out_shape=jax.ShapeDtypeStruct((B, H, N, D), jnp.float32),
    )(Q, K, V, idx_k, idx_q)

    return X


# speedup vs baseline: 1.1664x; 1.1664x over previous
import functools
import math

import jax
import jax.numpy as jnp
from jax import lax
from jax.experimental import pallas as pl
from jax.experimental.pallas import tpu as pltpu
from jax.experimental.pallas import tpu_sc as plsc

SEL = 64
N_ITER = 4
NEG = -3.0e38


def _sums_body(q_ref, k_ref, out_ref):
    ones = jnp.ones((64, 64), jnp.float32)
    ks = jax.lax.dot_general(ones, k_ref[0, 0], (((1,), (1,)), ((), ())),
                             precision=jax.lax.Precision.HIGH,
                             preferred_element_type=jnp.float32)
    qs = jax.lax.dot_general(ones, q_ref[0, 0], (((1,), (1,)), ((), ())),
                             precision=jax.lax.Precision.HIGH,
                             preferred_element_type=jnp.float32)
    out_ref[0, 0, 0, :] = ks[0]
    out_ref[0, 0, 1, :] = qs[0]


def _select_body(s_ref, idx_ref):
    R, N = s_ref.shape
    vals0 = s_ref[...]
    lane = jax.lax.broadcasted_iota(jnp.int32, (R, N), 1)
    col = jax.lax.broadcasted_iota(jnp.int32, (R, SEL), 1)

    def body(i, carry):
        vals, acc = carry
        idx = jnp.argmax(vals, axis=-1).astype(jnp.int32)
        acc = jnp.where(col == i, idx[:, None], acc)
        vals = jnp.where(lane == idx[:, None], NEG, vals)
        return vals, acc

    _, acc = jax.lax.fori_loop(
        0, SEL, body, (vals0, jnp.zeros((R, SEL), jnp.int32)))
    idx_ref[...] = acc


CH = 512
NCHUNK = 4096 // CH


def _sc_body(q_hbm, k_hbm, out_hbm, buf0, buf1, sums, orig, marr, lista,
             listb, final, sem0, sem1):
    w = lax.axis_index("s") * 2 + lax.axis_index("c")
    b = w // 16
    h = w % 16
    iota = lax.iota(jnp.int32, 16)
    fiota = iota.astype(jnp.float32) * 0.0
    bufs = (buf0, buf1)
    sems = (sem0, sem1)

    def chunk_sums(ch, buf):
        def row_body(r, _):
            p = ((buf[pl.ds(r * 64, 16)] + buf[pl.ds(r * 64 + 16, 16)])
                 + (buf[pl.ds(r * 64 + 32, 16)]
                    + buf[pl.ds(r * 64 + 48, 16)]))
            s = jnp.sum(p)
            plsc.store_scatter(
                sums, [jnp.broadcast_to(ch * CH + r, (16,))],
                jnp.broadcast_to(s, (16,)), mask=iota == 0)
            return 0

        lax.fori_loop(0, CH, row_body, 0)

    for src, hbm in ((0, k_hbm), (1, q_hbm)):
        cps = [None] * NCHUNK
        for ch in range(2):
            cps[ch] = pltpu.make_async_copy(
                hbm.at[b, h, pl.ds(ch * CH * 64, CH * 64)],
                bufs[ch % 2], sems[ch % 2])
            cps[ch].start()
        for ch in range(NCHUNK):
            cps[ch].wait()
            if ch + 2 < NCHUNK:
                cps[ch + 2] = pltpu.make_async_copy(
                    hbm.at[b, h, pl.ds((ch + 2) * CH * 64, CH * 64)],
                    bufs[ch % 2], sems[ch % 2])
            chunk_sums(ch, bufs[ch % 2])
            if ch + 2 < NCHUNK:
                cps[ch + 2].start()

        def copy_body(v, _):
            orig[pl.ds(v * 16, 16)] = sums[pl.ds(v * 16, 16)]
            return 0
        lax.fori_loop(0, 256, copy_body, 0)

        def mbuild(g, _):
            def mb(i, m):
                return jnp.maximum(m, sums[pl.ds(g * 256 + i * 16, 16)])
            marr[pl.ds(g * 16, 16)] = lax.fori_loop(0, 16, mb, fiota + NEG)
            return 0
        lax.fori_loop(0, 16, mbuild, 0)

        def extract(t, s_last):
            mv = [marr[pl.ds(g * 16, 16)] for g in range(16)]
            mm = mv[0]
            for g in range(1, 16):
                mm = jnp.maximum(mm, mv[g])
            gmax = jnp.max(mm)
            lsel = plsc.all_reduce_ffs(mm == gmax)
            colv = plsc.load_gather(marr, [iota * 16 + lsel])
            gsel = plsc.all_reduce_ffs(colv == gmax)
            cidx = gsel * 256 + iota * 16 + lsel
            gath = plsc.load_gather(sums, [cidx])
            isel = plsc.all_reduce_ffs(gath == gmax)
            gidx = gsel * 256 + isel * 16 + lsel
            plsc.store_scatter(sums, [jnp.broadcast_to(gidx, (16,))],
                               fiota + NEG, mask=iota == 0)
            gath2 = plsc.load_gather(sums, [cidx])
            newm = jnp.max(gath2)
            gsel_s = jnp.max(gsel)
            mrow = marr[pl.ds(gsel_s * 16, 16)]
            marr[pl.ds(gsel_s * 16, 16)] = jnp.where(iota == lsel, newm, mrow)
            return gmax

        s64 = lax.fori_loop(0, SEL, extract, jnp.float32(NEG))

        def gather_lists(v, carry):
            na, nb = carry
            x = orig[pl.ds(v * 16, 16)]
            idxv = v * 16 + iota
            gt = x > s64
            eq = x == s64
            plsc.store_compressed(lista.at[pl.ds(na, 16)], idxv, mask=gt)
            plsc.store_compressed(listb.at[pl.ds(nb, 16)], idxv, mask=eq)
            na = na + jnp.max(plsc.all_reduce_population_count(gt))
            nb = nb + jnp.max(plsc.all_reduce_population_count(eq))
            return na, nb

        n1, _ = lax.fori_loop(0, 256, gather_lists,
                              (jnp.int32(0), jnp.int32(0)))

        for g in range(4):
            final[pl.ds(g * 16, 16)] = lista[pl.ds(g * 16, 16)]
        for g in range(4):
            listbv = listb[pl.ds(g * 16, 16)]
            final[pl.ds(n1 + g * 16, 16)] = listbv

        pltpu.sync_copy(final.at[pl.ds(0, 128)], out_hbm.at[2 * w + src])


def _sc_select(Q, K):
    kern = pl.kernel(
        _sc_body,
        out_type=jax.ShapeDtypeStruct((2 * 32, 128), jnp.int32),
        mesh=plsc.VectorSubcoreMesh(core_axis_name="c", subcore_axis_name="s"),
        compiler_params=pltpu.CompilerParams(needs_layout_passes=False),
        scratch_types=[
            pltpu.VMEM((CH * 64,), jnp.float32),
            pltpu.VMEM((CH * 64,), jnp.float32),
            pltpu.VMEM((4096,), jnp.float32),
            pltpu.VMEM((4096,), jnp.float32),
            pltpu.VMEM((256,), jnp.float32),
            pltpu.VMEM((4160,), jnp.int32),
            pltpu.VMEM((4160,), jnp.int32),
            pltpu.VMEM((192,), jnp.int32),
            pltpu.SemaphoreType.DMA,
            pltpu.SemaphoreType.DMA,
        ],
    )
    B, H, N, D = Q.shape
    return kern(Q.reshape(B, H, N * D), K.reshape(B, H, N * D))[:, :SEL]


def _mm(a, b, ca, cb):
    return jax.lax.dot_general(
        a, b, ((( ca,), (cb,)), ((), ())),
        preferred_element_type=jnp.float32)


def _head(qs, k, v, idx_k, idx_q):
    N = qs.shape[0]
    lane = jax.lax.broadcasted_iota(jnp.int32, (SEL, N), 1)
    oh_k = (lane == idx_k[:, None]).astype(jnp.float32)
    oh_q = (lane == idx_q[:, None]).astype(jnp.float32)

    nc = _mm(oh_k, k, 1, 0)
    nr = _mm(oh_q, qs, 1, 0)

    c = _mm(qs, nc, 1, 1)
    E = jnp.exp(c - jnp.max(c, axis=-1, keepdims=True))
    S = jnp.sum(E, axis=-1, keepdims=True)

    cu = _mm(nr, nc, 1, 1)
    eu = jnp.exp(cu - jnp.max(cu, axis=-1, keepdims=True))
    u = eu / jnp.sum(eu, axis=-1, keepdims=True)

    r = _mm(nr, k, 1, 1)
    er = jnp.exp(r - jnp.max(r, axis=-1, keepdims=True))
    k3 = er / jnp.sum(er, axis=-1, keepdims=True)

    rv = _mm(k3, v, 1, 0)

    eye = (jax.lax.broadcasted_iota(jnp.int32, (SEL, SEL), 0)
           == jax.lax.broadcasted_iota(jnp.int32, (SEL, SEL), 1)
           ).astype(jnp.float32)
    denom = jnp.max(jnp.sum(u, axis=0))
    vinv = _mm(u, eye, 0, 0) * (1.0 / denom)
    for _ in range(N_ITER):
        kv = _mm(u, vinv, 1, 0)
        a1 = 7.0 * eye - kv
        a2 = 15.0 * eye - _mm(kv, a1, 1, 0)
        a3 = 13.0 * eye - _mm(kv, a2, 1, 0)
        vinv = 0.25 * _mm(vinv, a3, 1, 0)

    m = _mm(vinv, rv, 1, 0)
    return _mm(E, m, 1, 0) / S


def _main_body(q_ref, k_ref, v_ref, ik_ref, iq_ref, out_ref):
    scale = jnp.float32(1.0 / math.sqrt(64.0))
    for j in range(q_ref.shape[1]):
        out_ref[0, j] = _head(q_ref[0, j] * scale, k_ref[0, j], v_ref[0, j],
                              ik_ref[0, j], iq_ref[0, j])


HB = 2


def kernel(Q, K, V, mask):
    B, H, N, D = Q.shape
    BH = B * H

    idx = _sc_select(Q, K)

    idx = idx.reshape(B * H // HB, HB, 2, SEL)
    idx_k = idx[:, :, 0, :]
    idx_q = idx[:, :, 1, :]

    HBLK = H // HB
    X = pl.pallas_call(
        _main_body,
        grid=(BH // HB,),
        in_specs=[
            pl.BlockSpec((1, HB, N, D), lambda i: (i // HBLK, i % HBLK, 0, 0)),
            pl.BlockSpec((1, HB, N, D), lambda i: (i // HBLK, i % HBLK, 0, 0)),
            pl.BlockSpec((1, HB, N, D), lambda i: (i // HBLK, i % HBLK, 0, 0)),
            pl.BlockSpec((1, HB, SEL), lambda i: (i, 0, 0)),
            pl.BlockSpec((1, HB, SEL), lambda i: (i, 0, 0)),
        ],
        out_specs=pl.BlockSpec((1, HB, N, D), lambda i: (i // HBLK, i % HBLK, 0, 0)),
        out_shape=jax.ShapeDtypeStruct((B, H, N, D), jnp.float32),
    )(Q, K, V, idx_k, idx_q)

    return X


# speedup vs baseline: 1.3522x; 1.1593x over previous
import functools
import math

import jax
import jax.numpy as jnp
from jax import lax
from jax.experimental import pallas as pl
from jax.experimental.pallas import tpu as pltpu
from jax.experimental.pallas import tpu_sc as plsc

SEL = 64
N_ITER = 4
NEG = -3.0e38


def _sums_body(q_ref, k_ref, out_ref):
    ones = jnp.ones((64, 64), jnp.float32)
    ks = jax.lax.dot_general(ones, k_ref[0, 0], (((1,), (1,)), ((), ())),
                             precision=jax.lax.Precision.HIGH,
                             preferred_element_type=jnp.float32)
    qs = jax.lax.dot_general(ones, q_ref[0, 0], (((1,), (1,)), ((), ())),
                             precision=jax.lax.Precision.HIGH,
                             preferred_element_type=jnp.float32)
    out_ref[0, 0, 0, :] = ks[0]
    out_ref[0, 0, 1, :] = qs[0]


def _select_body(s_ref, idx_ref):
    R, N = s_ref.shape
    vals0 = s_ref[...]
    lane = jax.lax.broadcasted_iota(jnp.int32, (R, N), 1)
    col = jax.lax.broadcasted_iota(jnp.int32, (R, SEL), 1)

    def body(i, carry):
        vals, acc = carry
        idx = jnp.argmax(vals, axis=-1).astype(jnp.int32)
        acc = jnp.where(col == i, idx[:, None], acc)
        vals = jnp.where(lane == idx[:, None], NEG, vals)
        return vals, acc

    _, acc = jax.lax.fori_loop(
        0, SEL, body, (vals0, jnp.zeros((R, SEL), jnp.int32)))
    idx_ref[...] = acc


CH = 512
NCHUNK = 4096 // CH


def _sc_body(q_hbm, k_hbm, out_hbm, buf0, buf1, sums, orig, marr, lista,
             listb, final, sem0, sem1):
    w = lax.axis_index("s") * 2 + lax.axis_index("c")
    b = w // 16
    h = w % 16
    iota = lax.iota(jnp.int32, 16)
    fiota = iota.astype(jnp.float32) * 0.0
    bufs = (buf0, buf1)
    sems = (sem0, sem1)

    def chunk_sums(ch, buf):
        def row_body(r8, _):
            ss = []
            for t in range(8):
                r = r8 * 8 + t
                p = ((buf[pl.ds(r * 64, 16)] + buf[pl.ds(r * 64 + 16, 16)])
                     + (buf[pl.ds(r * 64 + 32, 16)]
                        + buf[pl.ds(r * 64 + 48, 16)]))
                ss.append(jnp.sum(p))
            for t in range(8):
                plsc.store_scatter(
                    sums, [jnp.broadcast_to(ch * CH + r8 * 8 + t, (16,))],
                    jnp.broadcast_to(ss[t], (16,)), mask=iota == 0)
            return 0

        lax.fori_loop(0, CH // 8, row_body, 0)

    for src, hbm in ((0, k_hbm), (1, q_hbm)):
        cps = [None] * NCHUNK
        for ch in range(2):
            cps[ch] = pltpu.make_async_copy(
                hbm.at[b, h, pl.ds(ch * CH * 64, CH * 64)],
                bufs[ch % 2], sems[ch % 2])
            cps[ch].start()
        for ch in range(NCHUNK):
            cps[ch].wait()
            if ch + 2 < NCHUNK:
                cps[ch + 2] = pltpu.make_async_copy(
                    hbm.at[b, h, pl.ds((ch + 2) * CH * 64, CH * 64)],
                    bufs[ch % 2], sems[ch % 2])
            chunk_sums(ch, bufs[ch % 2])
            if ch + 2 < NCHUNK:
                cps[ch + 2].start()

        def copy_body(v, _):
            orig[pl.ds(v * 16, 16)] = sums[pl.ds(v * 16, 16)]
            return 0
        lax.fori_loop(0, 256, copy_body, 0)

        def mbuild(g, _):
            def mb(i, m):
                return jnp.maximum(m, sums[pl.ds(g * 256 + i * 16, 16)])
            marr[pl.ds(g * 16, 16)] = lax.fori_loop(0, 16, mb, fiota + NEG)
            return 0
        lax.fori_loop(0, 16, mbuild, 0)

        def extract(t, s_last):
            mv = [marr[pl.ds(g * 16, 16)] for g in range(16)]
            mm = mv[0]
            for g in range(1, 16):
                mm = jnp.maximum(mm, mv[g])
            gmax = jnp.max(mm)
            lsel = plsc.all_reduce_ffs(mm == gmax)
            colv = plsc.load_gather(marr, [iota * 16 + lsel])
            gsel = plsc.all_reduce_ffs(colv == gmax)
            cidx = gsel * 256 + iota * 16 + lsel
            gath = plsc.load_gather(sums, [cidx])
            isel = plsc.all_reduce_ffs(gath == gmax)
            gidx = gsel * 256 + isel * 16 + lsel
            plsc.store_scatter(sums, [jnp.broadcast_to(gidx, (16,))],
                               fiota + NEG, mask=iota == 0)
            gath2 = plsc.load_gather(sums, [cidx])
            newm = jnp.max(gath2)
            gsel_s = jnp.max(gsel)
            mrow = marr[pl.ds(gsel_s * 16, 16)]
            marr[pl.ds(gsel_s * 16, 16)] = jnp.where(iota == lsel, newm, mrow)
            return gmax

        s64 = lax.fori_loop(0, SEL, extract, jnp.float32(NEG))

        def gather_lists(v, carry):
            na, nb = carry
            x = orig[pl.ds(v * 16, 16)]
            idxv = v * 16 + iota
            gt = x > s64
            eq = x == s64
            plsc.store_compressed(lista.at[pl.ds(na, 16)], idxv, mask=gt)
            plsc.store_compressed(listb.at[pl.ds(nb, 16)], idxv, mask=eq)
            na = na + jnp.max(plsc.all_reduce_population_count(gt))
            nb = nb + jnp.max(plsc.all_reduce_population_count(eq))
            return na, nb

        n1, _ = lax.fori_loop(0, 256, gather_lists,
                              (jnp.int32(0), jnp.int32(0)))

        for g in range(4):
            final[pl.ds(g * 16, 16)] = lista[pl.ds(g * 16, 16)]
        for g in range(4):
            listbv = listb[pl.ds(g * 16, 16)]
            final[pl.ds(n1 + g * 16, 16)] = listbv

        pltpu.sync_copy(final.at[pl.ds(0, 128)], out_hbm.at[2 * w + src])


def _sc_select(Q, K):
    kern = pl.kernel(
        _sc_body,
        out_type=jax.ShapeDtypeStruct((2 * 32, 128), jnp.int32),
        mesh=plsc.VectorSubcoreMesh(core_axis_name="c", subcore_axis_name="s"),
        compiler_params=pltpu.CompilerParams(needs_layout_passes=False),
        scratch_types=[
            pltpu.VMEM((CH * 64,), jnp.float32),
            pltpu.VMEM((CH * 64,), jnp.float32),
            pltpu.VMEM((4096,), jnp.float32),
            pltpu.VMEM((4096,), jnp.float32),
            pltpu.VMEM((256,), jnp.float32),
            pltpu.VMEM((4160,), jnp.int32),
            pltpu.VMEM((4160,), jnp.int32),
            pltpu.VMEM((192,), jnp.int32),
            pltpu.SemaphoreType.DMA,
            pltpu.SemaphoreType.DMA,
        ],
    )
    B, H, N, D = Q.shape
    return kern(Q.reshape(B, H, N * D), K.reshape(B, H, N * D))[:, :SEL]


def _mm(a, b, ca, cb):
    return jax.lax.dot_general(
        a, b, ((( ca,), (cb,)), ((), ())),
        preferred_element_type=jnp.float32)


def _head(qs, k, v, idx_k, idx_q):
    N = qs.shape[0]
    lane = jax.lax.broadcasted_iota(jnp.int32, (SEL, N), 1)
    oh_k = (lane == idx_k[:, None]).astype(jnp.float32)
    oh_q = (lane == idx_q[:, None]).astype(jnp.float32)

    nc = _mm(oh_k, k, 1, 0)
    nr = _mm(oh_q, qs, 1, 0)

    c = _mm(qs, nc, 1, 1)
    E = jnp.exp(c - jnp.max(c, axis=-1, keepdims=True))
    S = jnp.sum(E, axis=-1, keepdims=True)

    cu = _mm(nr, nc, 1, 1)
    eu = jnp.exp(cu - jnp.max(cu, axis=-1, keepdims=True))
    u = eu / jnp.sum(eu, axis=-1, keepdims=True)

    r = _mm(nr, k, 1, 1)
    er = jnp.exp(r - jnp.max(r, axis=-1, keepdims=True))
    k3 = er / jnp.sum(er, axis=-1, keepdims=True)

    rv = _mm(k3, v, 1, 0)

    eye = (jax.lax.broadcasted_iota(jnp.int32, (SEL, SEL), 0)
           == jax.lax.broadcasted_iota(jnp.int32, (SEL, SEL), 1)
           ).astype(jnp.float32)
    denom = jnp.max(jnp.sum(u, axis=0))
    vinv = _mm(u, eye, 0, 0) * (1.0 / denom)
    for _ in range(N_ITER):
        kv = _mm(u, vinv, 1, 0)
        a1 = 7.0 * eye - kv
        a2 = 15.0 * eye - _mm(kv, a1, 1, 0)
        a3 = 13.0 * eye - _mm(kv, a2, 1, 0)
        vinv = 0.25 * _mm(vinv, a3, 1, 0)

    m = _mm(vinv, rv, 1, 0)
    return _mm(E, m, 1, 0) / S


def _main_body(q_ref, k_ref, v_ref, ik_ref, iq_ref, out_ref):
    scale = jnp.float32(1.0 / math.sqrt(64.0))
    for j in range(q_ref.shape[1]):
        out_ref[0, j] = _head(q_ref[0, j] * scale, k_ref[0, j], v_ref[0, j],
                              ik_ref[0, j], iq_ref[0, j])


HB = 2


def kernel(Q, K, V, mask):
    B, H, N, D = Q.shape
    BH = B * H

    idx = _sc_select(Q, K)

    idx = idx.reshape(B * H // HB, HB, 2, SEL)
    idx_k = idx[:, :, 0, :]
    idx_q = idx[:, :, 1, :]

    HBLK = H // HB
    X = pl.pallas_call(
        _main_body,
        grid=(BH // HB,),
        in_specs=[
            pl.BlockSpec((1, HB, N, D), lambda i: (i // HBLK, i % HBLK, 0, 0)),
            pl.BlockSpec((1, HB, N, D), lambda i: (i // HBLK, i % HBLK, 0, 0)),
            pl.BlockSpec((1, HB, N, D), lambda i: (i // HBLK, i % HBLK, 0, 0)),
            pl.BlockSpec((1, HB, SEL), lambda i: (i, 0, 0)),
            pl.BlockSpec((1, HB, SEL), lambda i: (i, 0, 0)),
        ],
        out_specs=pl.BlockSpec((1, HB, N, D), lambda i: (i // HBLK, i % HBLK, 0, 0)),
        out_shape=jax.ShapeDtypeStruct((B, H, N, D), jnp.float32),
    )(Q, K, V, idx_k, idx_q)

    return X
